# baseline (speedup 1.0000x reference)
import functools

import jax
import jax.numpy as jnp
from jax import lax
from jax.experimental import pallas as pl
from jax.experimental.pallas import tpu as pltpu
from jax.experimental.pallas import tpu_sc as plsc

N_NODES = 10000
N_EDGES = 320000
D_FEAT = 128

NUM_CORES = 2
NUM_SUBCORES = 16
NUM_WORKERS = NUM_CORES * NUM_SUBCORES
EDGES_PER_WORKER = N_EDGES // NUM_WORKERS
CHUNK = 80
NUM_CHUNKS = EDGES_PER_WORKER // CHUNK
LANES = 16
GROUPS = CHUNK // LANES


def _make_sc_kernel():
    mesh = plsc.VectorSubcoreMesh(core_axis_name="c", subcore_axis_name="s")

    @functools.partial(
        pl.kernel,
        mesh=mesh,
        compiler_params=pltpu.CompilerParams(needs_layout_passes=False,
                                             use_tc_tiling_on_sc=False),
        out_type=jax.ShapeDtypeStruct((N_EDGES,), jnp.float32),
        scratch_types=[
            pltpu.VMEM((EDGES_PER_WORKER,), jnp.int32),
            pltpu.VMEM((EDGES_PER_WORKER,), jnp.int32),
            pltpu.VMEM((2, CHUNK, D_FEAT // 2), jnp.int32),
            pltpu.VMEM((2, CHUNK, D_FEAT // 2), jnp.int32),
            pltpu.VMEM((2, CHUNK), jnp.float32),
            pltpu.SemaphoreType.DMA,
            pltpu.SemaphoreType.DMA,
            pltpu.SemaphoreType.DMA,
            pltpu.SemaphoreType.DMA,
        ],
    )
    def sc_kernel(z_hbm, rows_hbm, cols_hbm, out_hbm,
                  ridx, cidx, rbuf, cbuf, obuf,
                  gsem0, gsem1, osem0, osem1):
        wid = lax.axis_index("s") * NUM_CORES + lax.axis_index("c")
        base = wid * EDGES_PER_WORKER

        cp_r = pltpu.async_copy(rows_hbm.at[pl.ds(base, EDGES_PER_WORKER)],
                                ridx, gsem0)
        cp_c = pltpu.async_copy(cols_hbm.at[pl.ds(base, EDGES_PER_WORKER)],
                                cidx, gsem0)
        cp_r.wait()
        cp_c.wait()

        def issue_gathers(chunk_id, buf, gsem):
            off = chunk_id * CHUNK
            pltpu.async_copy(z_hbm.at[ridx.at[pl.ds(off, CHUNK)]],
                             rbuf.at[buf], gsem)
            pltpu.async_copy(z_hbm.at[cidx.at[pl.ds(off, CHUNK)]],
                             cbuf.at[buf], gsem)

        def drain_gathers(buf, gsem):
            pltpu.make_async_copy(z_hbm.at[ridx.at[pl.ds(0, CHUNK)]],
                                  rbuf.at[buf], gsem).wait()
            pltpu.make_async_copy(z_hbm.at[cidx.at[pl.ds(0, CHUNK)]],
                                  cbuf.at[buf], gsem).wait()

        issue_gathers(0, 0, gsem0)

        def chunk_body(i, carry):
            p = lax.rem(i, 2)

            @pl.when(i + 1 < NUM_CHUNKS)
            def _():
                @pl.when(p == 0)
                def _():
                    issue_gathers(i + 1, 1, gsem1)

                @pl.when(p == 1)
                def _():
                    issue_gathers(i + 1, 0, gsem0)

            @pl.when(p == 0)
            def _():
                drain_gathers(0, gsem0)

            @pl.when(p == 1)
            def _():
                drain_gathers(1, gsem1)

            @pl.when(i >= 2)
            def _():
                @pl.when(p == 0)
                def _():
                    pltpu.make_async_copy(
                        obuf.at[0], out_hbm.at[pl.ds(base, CHUNK)],
                        osem0).wait()

                @pl.when(p == 1)
                def _():
                    pltpu.make_async_copy(
                        obuf.at[1], out_hbm.at[pl.ds(base, CHUNK)],
                        osem1).wait()

            lane = lax.iota(jnp.int32, LANES)
            last_lane = lane == (LANES - 1)

            def group_body(g, c2):
                gbase = g * LANES
                for j in range(LANES):
                    e = gbase + j
                    prods = []
                    for q in range(D_FEAT // (2 * LANES)):
                        ru = plsc.bitcast(
                            rbuf[p, e, pl.ds(q * LANES, LANES)],
                            jnp.bfloat16)
                        cu = plsc.bitcast(
                            cbuf[p, e, pl.ds(q * LANES, LANES)],
                            jnp.bfloat16)
                        prods.append(ru * cu)
                    s01 = prods[0] + prods[1]
                    s23 = prods[2] + prods[3]
                    sall = s01 + s23
                    a, b = plsc.unpack(
                        sall, format=plsc.PackFormat.INTERLEAVED)
                    cum = plsc.cumsum(a + b)
                    plsc.store_scatter(obuf.at[p],
                                       [jnp.full((LANES,), 0, jnp.int32) + e],
                                       cum, mask=last_lane)
                return c2

            lax.fori_loop(0, GROUPS, group_body, 0)

            off = base + i * CHUNK

            @pl.when(p == 0)
            def _():
                pltpu.async_copy(obuf.at[0], out_hbm.at[pl.ds(off, CHUNK)],
                                 osem0)

            @pl.when(p == 1)
            def _():
                pltpu.async_copy(obuf.at[1], out_hbm.at[pl.ds(off, CHUNK)],
                                 osem1)

            return carry

        lax.fori_loop(0, NUM_CHUNKS, chunk_body, 0)

        pltpu.make_async_copy(obuf.at[0], out_hbm.at[pl.ds(base, CHUNK)],
                              osem0).wait()
        pltpu.make_async_copy(obuf.at[1], out_hbm.at[pl.ds(base, CHUNK)],
                              osem1).wait()

    return sc_kernel


_sc_kernel = _make_sc_kernel()


def kernel(z, edge_index):
    rows = edge_index[0].astype(jnp.int32)
    cols = edge_index[1].astype(jnp.int32)
    zb = z.astype(jnp.bfloat16)
    z32 = lax.bitcast_convert_type(
        zb.reshape(N_NODES, D_FEAT // 2, 2), jnp.int32)
    return _sc_kernel(z32, rows, cols)


# speedup vs baseline: 1.5143x; 1.5143x over previous
import functools

import jax
import jax.numpy as jnp
from jax import lax
from jax.experimental import pallas as pl
from jax.experimental.pallas import tpu as pltpu
from jax.experimental.pallas import tpu_sc as plsc

N_NODES = 10000
N_EDGES = 320000
D_FEAT = 128

NUM_CORES = 2
NUM_SUBCORES = 16
NUM_WORKERS = NUM_CORES * NUM_SUBCORES
EDGES_PER_WORKER = N_EDGES // NUM_WORKERS
CHUNK = 80
NUM_CHUNKS = EDGES_PER_WORKER // CHUNK
LANES = 16
GROUPS = CHUNK // LANES


def _make_sc_kernel():
    mesh = plsc.VectorSubcoreMesh(core_axis_name="c", subcore_axis_name="s")

    @functools.partial(
        pl.kernel,
        mesh=mesh,
        compiler_params=pltpu.CompilerParams(needs_layout_passes=False,
                                             use_tc_tiling_on_sc=False),
        out_type=jax.ShapeDtypeStruct((N_EDGES,), jnp.float32),
        scratch_types=[
            pltpu.VMEM((EDGES_PER_WORKER,), jnp.int32),
            pltpu.VMEM((EDGES_PER_WORKER,), jnp.int32),
            pltpu.VMEM((2, CHUNK, D_FEAT // 2), jnp.int32),
            pltpu.VMEM((2, CHUNK, D_FEAT // 2), jnp.int32),
            pltpu.VMEM((2, CHUNK), jnp.float32),
            pltpu.SemaphoreType.DMA,
            pltpu.SemaphoreType.DMA,
            pltpu.SemaphoreType.DMA,
            pltpu.SemaphoreType.DMA,
        ],
    )
    def sc_kernel(z_hbm, rows_hbm, cols_hbm, out_hbm,
                  ridx, cidx, rbuf, cbuf, obuf,
                  gsem0, gsem1, osem0, osem1):
        wid = lax.axis_index("s") * NUM_CORES + lax.axis_index("c")
        base = wid * EDGES_PER_WORKER

        cp_r = pltpu.async_copy(rows_hbm.at[pl.ds(base, EDGES_PER_WORKER)],
                                ridx, gsem0)
        cp_c = pltpu.async_copy(cols_hbm.at[pl.ds(base, EDGES_PER_WORKER)],
                                cidx, gsem0)
        cp_r.wait()
        cp_c.wait()

        def issue_gathers(chunk_id, buf, gsem):
            off = chunk_id * CHUNK
            pltpu.async_copy(z_hbm.at[ridx.at[pl.ds(off, CHUNK)]],
                             rbuf.at[buf], gsem)
            pltpu.async_copy(z_hbm.at[cidx.at[pl.ds(off, CHUNK)]],
                             cbuf.at[buf], gsem)

        def drain_gathers(buf, gsem):
            pltpu.make_async_copy(z_hbm.at[ridx.at[pl.ds(0, CHUNK)]],
                                  rbuf.at[buf], gsem).wait()
            pltpu.make_async_copy(z_hbm.at[cidx.at[pl.ds(0, CHUNK)]],
                                  cbuf.at[buf], gsem).wait()

        issue_gathers(0, 0, gsem0)

        def chunk_body(i, carry):
            p = lax.rem(i, 2)

            @pl.when(i + 1 < NUM_CHUNKS)
            def _():
                @pl.when(p == 0)
                def _():
                    issue_gathers(i + 1, 1, gsem1)

                @pl.when(p == 1)
                def _():
                    issue_gathers(i + 1, 0, gsem0)

            @pl.when(p == 0)
            def _():
                drain_gathers(0, gsem0)

            @pl.when(p == 1)
            def _():
                drain_gathers(1, gsem1)

            @pl.when(i >= 2)
            def _():
                @pl.when(p == 0)
                def _():
                    pltpu.make_async_copy(
                        obuf.at[0], out_hbm.at[pl.ds(base, CHUNK)],
                        osem0).wait()

                @pl.when(p == 1)
                def _():
                    pltpu.make_async_copy(
                        obuf.at[1], out_hbm.at[pl.ds(base, CHUNK)],
                        osem1).wait()

            lane = lax.iota(jnp.int32, LANES)

            def group_body(g, c2):
                gbase = g * LANES
                res = jnp.zeros((LANES,), jnp.float32)
                for j in range(LANES):
                    e = gbase + j
                    prods = []
                    for q in range(D_FEAT // (2 * LANES)):
                        ru = plsc.bitcast(
                            rbuf[p, e, pl.ds(q * LANES, LANES)],
                            jnp.bfloat16)
                        cu = plsc.bitcast(
                            cbuf[p, e, pl.ds(q * LANES, LANES)],
                            jnp.bfloat16)
                        prods.append(ru * cu)
                    s01 = prods[0] + prods[1]
                    s23 = prods[2] + prods[3]
                    sall = s01 + s23
                    a, b = plsc.unpack(
                        sall, format=plsc.PackFormat.INTERLEAVED)
                    s = plsc.cumsum(a + b)[LANES - 1]
                    res = jnp.where(lane == j, s, res)
                obuf[p, pl.ds(gbase, LANES)] = res
                return c2

            lax.fori_loop(0, GROUPS, group_body, 0)

            off = base + i * CHUNK

            @pl.when(p == 0)
            def _():
                pltpu.async_copy(obuf.at[0], out_hbm.at[pl.ds(off, CHUNK)],
                                 osem0)

            @pl.when(p == 1)
            def _():
                pltpu.async_copy(obuf.at[1], out_hbm.at[pl.ds(off, CHUNK)],
                                 osem1)

            return carry

        lax.fori_loop(0, NUM_CHUNKS, chunk_body, 0)

        pltpu.make_async_copy(obuf.at[0], out_hbm.at[pl.ds(base, CHUNK)],
                              osem0).wait()
        pltpu.make_async_copy(obuf.at[1], out_hbm.at[pl.ds(base, CHUNK)],
                              osem1).wait()

    return sc_kernel


_sc_kernel = _make_sc_kernel()


def kernel(z, edge_index):
    rows = edge_index[0].astype(jnp.int32)
    cols = edge_index[1].astype(jnp.int32)
    zb = z.astype(jnp.bfloat16)
    z32 = lax.bitcast_convert_type(
        zb.reshape(N_NODES, D_FEAT // 2, 2), jnp.int32)
    return _sc_kernel(z32, rows, cols)
